# Initial kernel scaffold; baseline (speedup 1.0000x reference)
import jax
import jax.numpy as jnp
from jax.experimental import pallas as pl


def kernel(x, edge_index, lstm_data, W1, b1, W2, b2, Wih0, Whh0, bih0, bhh0, Wih1, Whh1, bih1, bhh1, lw1, lb1, lw2, lb2, lw3, lb3):
    raise NotImplementedError("write your pallas kernel here")


# speedup vs baseline: 10.5588x; 10.5588x over previous
import functools

import jax
import jax.numpy as jnp
from jax import lax
from jax.experimental import pallas as pl
from jax.experimental.pallas import tpu as pltpu
from jax.experimental.pallas import tpu_sc as plsc

N_NODES = 10000
FEAT = 128
N_EDGES = 320000
T_STEPS = 20
IN_SZ = 16
HID = 64

CHUNK = 128
NUM_CORES = 2
NUM_SUBCORES = 16
NW = NUM_CORES * NUM_SUBCORES
EDGE_ROWS = 2528
ROWS_PER_WORKER = EDGE_ROWS // NW
ACC_ROWS = 10240
ROWS_PER_TILE = ACC_ROWS // NUM_SUBCORES
ZCHUNKS = ROWS_PER_TILE // CHUNK
TRASH_ROW = 10200
DEG_COLS = 16

BLK = 1000
GRID = N_NODES // BLK

_MESH = plsc.VectorSubcoreMesh(
    core_axis_name="c", subcore_axis_name="s",
    num_cores=NUM_CORES, num_subcores=NUM_SUBCORES)


def _fill(ref, nrows, ncols, val):
    nc16 = ncols // 16

    def row(i, c):
        def col(j, c2):
            ref[i, pl.ds(j * 16, 16)] = jnp.full((16,), val, jnp.float32)
            return c2
        return lax.fori_loop(0, nc16, col, c)

    lax.fori_loop(0, nrows, row, 0)


@functools.partial(
    pl.kernel,
    out_type=jax.ShapeDtypeStruct((NUM_CORES, ACC_ROWS, DEG_COLS), jnp.float32),
    mesh=_MESH,
    scratch_types=[
        pltpu.VMEM((CHUNK,), jnp.int32),
        pltpu.VMEM((CHUNK, DEG_COLS), jnp.float32),
        pltpu.VMEM_SHARED((ACC_ROWS, DEG_COLS), jnp.float32),
    ],
)
def _sc_degree(dst_rows, out, idx_v, buf_v, acc):
    cid = lax.axis_index("c")
    sid = lax.axis_index("s")
    wid = sid * NUM_CORES + cid
    base = sid * ROWS_PER_TILE

    _fill(buf_v, CHUNK, DEG_COLS, 0.0)
    for k in range(ZCHUNKS):
        pltpu.sync_copy(buf_v, acc.at[pl.ds(base + k * CHUNK, CHUNK)])
    _fill(buf_v, CHUNK, DEG_COLS, 1.0)
    plsc.subcore_barrier()

    def step(j, c):
        row = wid * ROWS_PER_WORKER + j
        pltpu.sync_copy(dst_rows.at[row], idx_v)
        pltpu.sync_copy(buf_v, acc.at[idx_v], add=True)
        return c

    lax.fori_loop(0, ROWS_PER_WORKER, step, 0)
    plsc.subcore_barrier()

    for k in range(ZCHUNKS):
        pltpu.sync_copy(acc.at[pl.ds(base + k * CHUNK, CHUNK)], buf_v)
        pltpu.sync_copy(buf_v, out.at[cid, pl.ds(base + k * CHUNK, CHUNK)])


@functools.partial(
    pl.kernel,
    out_type=jax.ShapeDtypeStruct((NUM_CORES, ACC_ROWS, FEAT), jnp.float32),
    mesh=_MESH,
    scratch_types=[
        pltpu.VMEM((CHUNK,), jnp.int32),
        pltpu.VMEM((CHUNK,), jnp.int32),
        pltpu.VMEM((CHUNK, FEAT), jnp.float32),
        pltpu.VMEM_SHARED((ACC_ROWS, FEAT), jnp.float32),
        pltpu.SemaphoreType.DMA,
    ],
)
def _sc_segsum(y, src_rows, dst_rows, out, sidx, didx, rows_v, acc, sem):
    cid = lax.axis_index("c")
    sid = lax.axis_index("s")
    wid = sid * NUM_CORES + cid
    base = sid * ROWS_PER_TILE

    _fill(rows_v, CHUNK, FEAT, 0.0)
    for k in range(ZCHUNKS):
        pltpu.sync_copy(rows_v, acc.at[pl.ds(base + k * CHUNK, CHUNK)])
    plsc.subcore_barrier()

    def step(j, c):
        row = wid * ROWS_PER_WORKER + j
        pltpu.sync_copy(src_rows.at[row], sidx)
        pltpu.sync_copy(dst_rows.at[row], didx)
        pltpu.async_copy(y.at[sidx], rows_v, sem).wait()
        pltpu.sync_copy(rows_v, acc.at[didx], add=True)
        return c

    lax.fori_loop(0, ROWS_PER_WORKER, step, 0)
    plsc.subcore_barrier()

    for k in range(ZCHUNKS):
        pltpu.sync_copy(acc.at[pl.ds(base + k * CHUNK, CHUNK)], rows_v)
        pltpu.sync_copy(rows_v, out.at[cid, pl.ds(base + k * CHUNK, CHUNK)])


def _dinv(degp):
    deg = degp[0][:, 0:1] + degp[1][:, 0:1] + 1.0
    return lax.rsqrt(deg)


def _tc_y_body(xg_ref, w_ref, degp_ref, y_ref):
    d = _dinv(degp_ref)
    y_ref[...] = jnp.dot(xg_ref[...], w_ref[...],
                         preferred_element_type=jnp.float32) * d


def _tc_combine_body(acc_ref, y1_ref, degp_ref, b_ref, w_ref, y2_ref):
    d = _dinv(degp_ref)
    s = acc_ref[0] + acc_ref[1] + y1_ref[...]
    h = jnp.maximum(d * s + b_ref[...], 0.0)
    y2_ref[...] = jnp.dot(h, w_ref[...], preferred_element_type=jnp.float32) * d


def _tc_final_body(acc_ref, y2_ref, degp_ref, b_ref, xt_ref, ext_ref,
                   wa_ref, wb_ref, wc_ref, lb1_ref, lw2_ref, lb2_ref,
                   lw3_ref, lb3_ref, out_ref):
    d = _dinv(degp_ref)
    s = acc_ref[0] + acc_ref[1] + y2_ref[...]
    xg2 = jnp.maximum(d * s + b_ref[...], 0.0)
    z = jnp.dot(xg2, wa_ref[...], preferred_element_type=jnp.float32)
    z += jnp.dot(jnp.maximum(xt_ref[...], 0.0), wb_ref[...],
                 preferred_element_type=jnp.float32)
    z += jnp.dot(jnp.maximum(ext_ref[...], 0.0), wc_ref[...],
                 preferred_element_type=jnp.float32)
    z = jnp.maximum(z + lb1_ref[...], 0.0)
    z = jnp.maximum(jnp.dot(z, lw2_ref[...],
                            preferred_element_type=jnp.float32) + lb2_ref[...], 0.0)
    out_ref[...] = jnp.dot(z, lw3_ref[...],
                           preferred_element_type=jnp.float32) + lb3_ref[...]


def _lstm_gates(g, c):
    i = jax.nn.sigmoid(g[:, 0:HID])
    f = jax.nn.sigmoid(g[:, HID:2 * HID])
    gg = jnp.tanh(g[:, 2 * HID:3 * HID])
    o = jax.nn.sigmoid(g[:, 3 * HID:4 * HID])
    c = f * c + i * gg
    return o * jnp.tanh(c), c


def _tc_lstm_body(xb_ref, w0x_ref, w0h_ref, b0_ref, w1x_ref, w1h_ref, b1_ref,
                  out_ref, hseq_ref):
    h = jnp.zeros((BLK, HID), jnp.float32)
    c = jnp.zeros((BLK, HID), jnp.float32)
    b0 = b0_ref[...]
    w0x = w0x_ref[...]
    w0h = w0h_ref[...]
    for t in range(T_STEPS):
        xt = xb_ref[:, t * IN_SZ:(t + 1) * IN_SZ]
        g = (jnp.dot(xt, w0x, preferred_element_type=jnp.float32)
             + jnp.dot(h, w0h, preferred_element_type=jnp.float32) + b0)
        h, c = _lstm_gates(g, c)
        hseq_ref[:, t * HID:(t + 1) * HID] = h
    h1 = jnp.zeros((BLK, HID), jnp.float32)
    c1 = jnp.zeros((BLK, HID), jnp.float32)
    b1 = b1_ref[...]
    w1x = w1x_ref[...]
    w1h = w1h_ref[...]
    for t in range(T_STEPS):
        xt = hseq_ref[:, t * HID:(t + 1) * HID]
        g = (jnp.dot(xt, w1x, preferred_element_type=jnp.float32)
             + jnp.dot(h1, w1h, preferred_element_type=jnp.float32) + b1)
        h1, c1 = _lstm_gates(g, c1)
    out_ref[...] = jnp.mean(h1, axis=1, keepdims=True)


def _full(shape):
    return pl.BlockSpec(shape, lambda i: tuple(0 for _ in shape))


_tc_y = pl.pallas_call(
    _tc_y_body,
    grid=(GRID,),
    in_specs=[
        pl.BlockSpec((BLK, FEAT), lambda i: (i, 0)),
        _full((FEAT, FEAT)),
        pl.BlockSpec((2, BLK, DEG_COLS), lambda i: (0, i, 0)),
    ],
    out_specs=pl.BlockSpec((BLK, FEAT), lambda i: (i, 0)),
    out_shape=jax.ShapeDtypeStruct((N_NODES, FEAT), jnp.float32),
)

_tc_combine = pl.pallas_call(
    _tc_combine_body,
    grid=(GRID,),
    in_specs=[
        pl.BlockSpec((2, BLK, FEAT), lambda i: (0, i, 0)),
        pl.BlockSpec((BLK, FEAT), lambda i: (i, 0)),
        pl.BlockSpec((2, BLK, DEG_COLS), lambda i: (0, i, 0)),
        _full((1, FEAT)),
        _full((FEAT, FEAT)),
    ],
    out_specs=pl.BlockSpec((BLK, FEAT), lambda i: (i, 0)),
    out_shape=jax.ShapeDtypeStruct((N_NODES, FEAT), jnp.float32),
)

_tc_final = pl.pallas_call(
    _tc_final_body,
    grid=(GRID,),
    in_specs=[
        pl.BlockSpec((2, BLK, FEAT), lambda i: (0, i, 0)),
        pl.BlockSpec((BLK, FEAT), lambda i: (i, 0)),
        pl.BlockSpec((2, BLK, DEG_COLS), lambda i: (0, i, 0)),
        _full((1, FEAT)),
        pl.BlockSpec((BLK, 1), lambda i: (i, 0)),
        pl.BlockSpec((BLK, 2), lambda i: (i, 0)),
        _full((FEAT, HID)),
        _full((1, HID)),
        _full((2, HID)),
        _full((1, HID)),
        _full((HID, HID)),
        _full((1, HID)),
        _full((HID, 1)),
        _full((1, 1)),
    ],
    out_specs=pl.BlockSpec((BLK, 1), lambda i: (i, 0)),
    out_shape=jax.ShapeDtypeStruct((N_NODES, 1), jnp.float32),
)

_tc_lstm = pl.pallas_call(
    _tc_lstm_body,
    grid=(GRID,),
    in_specs=[
        pl.BlockSpec((BLK, T_STEPS * IN_SZ), lambda i: (i, 0)),
        _full((IN_SZ, 4 * HID)),
        _full((HID, 4 * HID)),
        _full((1, 4 * HID)),
        _full((HID, 4 * HID)),
        _full((HID, 4 * HID)),
        _full((1, 4 * HID)),
    ],
    out_specs=pl.BlockSpec((BLK, 1), lambda i: (i, 0)),
    out_shape=jax.ShapeDtypeStruct((N_NODES, 1), jnp.float32),
    scratch_shapes=[pltpu.VMEM((BLK, T_STEPS * HID), jnp.float32)],
)


def kernel(x, edge_index, lstm_data, W1, b1, W2, b2, Wih0, Whh0, bih0, bhh0,
           Wih1, Whh1, bih1, bhh1, lw1, lb1, lw2, lb2, lw3, lb3):
    ext = x[:, 0:2]
    xg = x[:, 2:]
    src = edge_index[0].astype(jnp.int32)
    dst = edge_index[1].astype(jnp.int32)
    pad = EDGE_ROWS * CHUNK - N_EDGES
    srcp = jnp.concatenate([src, jnp.zeros((pad,), jnp.int32)]).reshape(
        EDGE_ROWS, CHUNK)
    dstp = jnp.concatenate([dst, jnp.full((pad,), TRASH_ROW, jnp.int32)]).reshape(
        EDGE_ROWS, CHUNK)

    degp = _sc_degree(dstp)
    y1 = _tc_y(xg, W1, degp)
    acc1 = _sc_segsum(y1, srcp, dstp)
    y2 = _tc_combine(acc1, y1, degp, b1[None, :], W2)
    acc2 = _sc_segsum(y2, srcp, dstp)

    xt = _tc_lstm(lstm_data.reshape(N_NODES, T_STEPS * IN_SZ),
                  Wih0.T, Whh0.T, (bih0 + bhh0)[None, :],
                  Wih1.T, Whh1.T, (bih1 + bhh1)[None, :])

    return _tc_final(acc2, y2, degp, b2[None, :], xt, ext,
                     lw1[:FEAT], lw1[FEAT:FEAT + 1], lw1[FEAT + 1:],
                     lb1[None, :], lw2, lb2[None, :], lw3, lb3[None, :])
